# baseline (speedup 1.0000x reference)
import functools
import math

import jax
import jax.numpy as jnp
from jax import lax
from jax.experimental import pallas as pl
from jax.experimental.pallas import tpu as pltpu
from jax.experimental.pallas import tpu_sc as plsc

N = 100000
E = 1600000
VOCAB = 101
D = 32
H = 32
OUT = 32
L = 6

_BN = 1.0 / math.sqrt(1.0 + 1e-5)

NC = 2
NS = 16
ROW = 128
CH = 3
NBUF = 4
G = 264
EPT = CH * ROW * G
E_PAD = EPT * NS
EB = E_PAD // ROW
NZ_PAD = 16 * 6272
ZPT = NZ_PAD // NS
ZCH = 896
OROWS = 2 * N + 256
SROWS = NZ_PAD // ROW
SPT = SROWS // NS


CR = CH * ROW


def _sc_agg_body(h2, src2, dst2, scat2, out, src_v, dst_v, rows_v, agg_sh,
                 g0, g1, g2, g3, s0, s1, s2, s3):
    c = lax.axis_index("c")
    s = lax.axis_index("s")
    gsem = (g0, g1, g2, g3)
    ssem = (s0, s1, s2, s3)

    def _zb(i, _):
        rows_v[i, :] = jnp.zeros((16,), jnp.float32)
        return 0
    lax.fori_loop(0, NBUF * CR, _zb, 0)
    z0 = s * ZPT
    for z in range(4):
        pltpu.sync_copy(rows_v.at[pl.ds(0, NBUF * CR)],
                        agg_sh.at[pl.ds(z0 + z * NBUF * CR, NBUF * CR)])
    pltpu.sync_copy(rows_v.at[pl.ds(0, ROW)],
                    agg_sh.at[pl.ds(z0 + 4 * NBUF * CR, ROW)])
    plsc.subcore_barrier()

    def load_idx(b, g):
        base = (s * G + g) * CH
        pltpu.sync_copy(src2.at[c, pl.ds(base, CH)], src_v.at[b])
        pltpu.sync_copy(dst2.at[pl.ds(base, CH)], dst_v.at[b])

    def fire_gathers(b):
        for j in range(CH):
            pltpu.async_copy(h2.at[src_v.at[b, j]],
                             rows_v.at[pl.ds(b * CR + j * ROW, ROW)],
                             gsem[b])

    def drain_gathers(b):
        pltpu.make_async_copy(h2.at[pl.ds(0, CR)],
                              rows_v.at[pl.ds(b * CR, CR)], gsem[b]).wait()

    def fire_scatters(b):
        for j in range(CH):
            pltpu.async_copy(rows_v.at[pl.ds(b * CR + j * ROW, ROW)],
                             agg_sh.at[dst_v.at[b, j]], ssem[b], add=True)

    def drain_scatters(b, rows=CR):
        pltpu.make_async_copy(h2.at[pl.ds(0, rows)],
                              agg_sh.at[pl.ds(0, rows)], ssem[b]).wait()

    def _eb(t, _):
        for k in range(NBUF):
            @pl.when(t > 0)
            def _(k=k):
                drain_scatters(k)
            load_idx(k, NBUF * t + k)
            fire_gathers(k)
            b2 = (k - 2) % NBUF
            if k < 2:
                @pl.when(t > 0)
                def _(b2=b2):
                    drain_gathers(b2)
                    fire_scatters(b2)
            else:
                drain_gathers(b2)
                fire_scatters(b2)
        return 0
    lax.fori_loop(0, G // NBUF, _eb, 0)

    drain_gathers(2)
    fire_scatters(2)
    drain_gathers(3)
    fire_scatters(3)
    for b in range(NBUF):
        drain_scatters(b)
    plsc.subcore_barrier()

    rb = s * ZPT
    ib = s * SPT
    for z in range(16):
        b = z % NBUF
        if z >= NBUF:
            drain_scatters(b)
        pltpu.sync_copy(scat2.at[c, pl.ds(ib + z * CH, CH)], src_v.at[b])
        pltpu.sync_copy(agg_sh.at[pl.ds(rb + z * CR, CR)],
                        rows_v.at[pl.ds(b * CR, CR)])
        for j in range(CH):
            pltpu.async_copy(rows_v.at[pl.ds(b * CR + j * ROW, ROW)],
                             out.at[src_v.at[b, j]], ssem[b])
    drain_scatters(0)
    pltpu.sync_copy(scat2.at[c, pl.ds(ib + 48, 1)], src_v.at[0, pl.ds(0, 1)])
    pltpu.sync_copy(agg_sh.at[pl.ds(rb + 48 * ROW, ROW)],
                    rows_v.at[pl.ds(0, ROW)])
    pltpu.async_copy(rows_v.at[pl.ds(0, ROW)],
                     out.at[src_v.at[0, 0]], ssem[0])
    drain_scatters(1)
    drain_scatters(2)
    drain_scatters(3)
    drain_scatters(0, ROW)


@functools.cache
def _sc_agg_kernel():
    return pl.kernel(
        _sc_agg_body,
        out_type=jax.ShapeDtypeStruct((OROWS, 16), jnp.float32),
        mesh=plsc.VectorSubcoreMesh(core_axis_name="c", subcore_axis_name="s",
                                    num_cores=NC, num_subcores=NS),
        compiler_params=pltpu.CompilerParams(use_tc_tiling_on_sc=False),
        scratch_types=[
            pltpu.VMEM((NBUF, CH, ROW), jnp.int32),
            pltpu.VMEM((NBUF, CH, ROW), jnp.int32),
            pltpu.VMEM((NBUF * CR, 16), jnp.float32),
            pltpu.VMEM_SHARED((NZ_PAD, 16), jnp.float32),
            pltpu.SemaphoreType.DMA,
            pltpu.SemaphoreType.DMA,
            pltpu.SemaphoreType.DMA,
            pltpu.SemaphoreType.DMA,
            pltpu.SemaphoreType.DMA,
            pltpu.SemaphoreType.DMA,
            pltpu.SemaphoreType.DMA,
            pltpu.SemaphoreType.DMA,
        ],
    )


def _sc_agg(h2, src2, dst2, scat2):
    return _sc_agg_kernel()(h2, src2, dst2, scat2)


NB = 20000
NB4 = NB // 4


def _embed_body(nt_ref, emb_ref, h_ref):
    ids = nt_ref[...]
    parts = []
    for sl in range(4):
        iota = lax.broadcasted_iota(jnp.int32, (NB4, 128), 1)
        oh = (ids[:, sl:sl + 1] == iota).astype(jnp.float32)
        parts.append(jnp.dot(oh, emb_ref[...],
                             preferred_element_type=jnp.float32))
    h_ref[...] = jnp.concatenate(parts, axis=1)


def _mlp_body(h_ref, agg_ref, w1_ref, b1_ref, w2_ref, b2_ref,
              hn_ref, pin_ref, pout_ref):
    i = pl.program_id(0)
    h = h_ref[...]
    rst = h + agg_ref[...]
    t = jnp.maximum(
        jnp.dot(rst, w1_ref[...], preferred_element_type=jnp.float32)
        + b1_ref[...], 0.0) * _BN
    m = jnp.dot(t, w2_ref[...], preferred_element_type=jnp.float32) \
        + b2_ref[...]
    y = jnp.maximum(m, 0.0) * (_BN * _BN)
    hn_ref[...] = y

    @pl.when(i == 0)
    def _():
        pin_ref[...] = jnp.zeros_like(pin_ref)
        pout_ref[...] = jnp.zeros_like(pout_ref)

    pin_ref[...] += jnp.sum(h, axis=0, keepdims=True)
    pout_ref[...] += jnp.sum(y, axis=0, keepdims=True)


IDXB = 1584


def _idx_body(src_ref, dst_ref, src2_ref, dst2_ref):
    i = pl.program_id(0)
    rows = lax.broadcasted_iota(jnp.int32, (IDXB, 128), 0) + i * IDXB
    cols = lax.broadcasted_iota(jnp.int32, (IDXB, 128), 1)
    gid = rows * 128 + cols
    valid = gid < E
    sv = jnp.where(valid, src_ref[0], gid & 0xFFF)
    src2_ref[0] = 2 * sv
    src2_ref[1] = 2 * sv + 1
    dst2_ref[...] = jnp.where(valid, dst_ref[0], N + (gid & 15))


_idx_call = pl.pallas_call(
    _idx_body,
    grid=(EB // IDXB,),
    in_specs=[
        pl.BlockSpec((1, IDXB, 128), lambda i: (0, i, 0)),
        pl.BlockSpec((1, IDXB, 128), lambda i: (1, i, 0)),
    ],
    out_specs=[
        pl.BlockSpec((NC, IDXB, 128), lambda i: (0, i, 0)),
        pl.BlockSpec((IDXB, 128), lambda i: (i, 0)),
    ],
    out_shape=[
        jax.ShapeDtypeStruct((NC, EB, 128), jnp.int32),
        jax.ShapeDtypeStruct((EB, 128), jnp.int32),
    ],
)


def _score_body(pf_ref, pw_ref, pb_ref, out_ref):
    out_ref[...] = (
        jnp.dot(pf_ref[...], pw_ref[...], preferred_element_type=jnp.float32)
        + jnp.sum(pb_ref[...], axis=0, keepdims=True))


_embed_call = pl.pallas_call(
    _embed_body,
    grid=(N // NB,),
    in_specs=[
        pl.BlockSpec((NB4, 4), lambda i: (i, 0)),
        pl.BlockSpec((128, D), lambda i: (0, 0)),
    ],
    out_specs=pl.BlockSpec((NB4, 128), lambda i: (i, 0)),
    out_shape=jax.ShapeDtypeStruct((N // 4, 128), jnp.float32),
)

_mlp_call = pl.pallas_call(
    _mlp_body,
    grid=(N // NB,),
    in_specs=[
        pl.BlockSpec((NB4, 128), lambda i: (i, 0)),
        pl.BlockSpec((NB4, 128), lambda i: (i, 0)),
        pl.BlockSpec((128, 128), lambda i: (0, 0)),
        pl.BlockSpec((1, 128), lambda i: (0, 0)),
        pl.BlockSpec((128, 128), lambda i: (0, 0)),
        pl.BlockSpec((1, 128), lambda i: (0, 0)),
    ],
    out_specs=[
        pl.BlockSpec((NB4, 128), lambda i: (i, 0)),
        pl.BlockSpec((1, 128), lambda i: (0, 0)),
        pl.BlockSpec((1, 128), lambda i: (0, 0)),
    ],
    out_shape=[
        jax.ShapeDtypeStruct((N // 4, 128), jnp.float32),
        jax.ShapeDtypeStruct((1, 128), jnp.float32),
        jax.ShapeDtypeStruct((1, 128), jnp.float32),
    ],
)

_score_call = pl.pallas_call(
    _score_body,
    in_specs=[
        pl.BlockSpec((1, L * 128), lambda: (0, 0)),
        pl.BlockSpec((L * 128, OUT), lambda: (0, 0)),
        pl.BlockSpec((L, OUT), lambda: (0, 0)),
    ],
    out_specs=pl.BlockSpec((1, OUT), lambda: (0, 0)),
    out_shape=jax.ShapeDtypeStruct((1, OUT), jnp.float32),
)


def kernel(node_type, edge_index, emb, gW1, gb1, gW2, gb2, pW, pb):
    e3 = edge_index.astype(jnp.int32).reshape(2, E // 128, 128)
    src2, dst2 = _idx_call(e3, e3)

    k = jnp.arange(NZ_PAD, dtype=jnp.int32)
    scat2 = jnp.stack(
        [jnp.where(k < N, 2 * k + c, 2 * N + (2 * k + c) % 256)
         for c in range(NC)]).reshape(NC, SROWS, ROW)

    emb_pad = jnp.zeros((128, D), jnp.float32).at[:VOCAB].set(emb)
    h = _embed_call(node_type.astype(jnp.int32).reshape(N // 4, 4), emb_pad)

    eye4 = jnp.eye(4, dtype=jnp.float32)
    pooled = []
    p0 = None
    for i in range(L - 1):
        agg = _sc_agg(h.reshape(2 * N, 16), src2, dst2, scat2)
        agg4 = agg.reshape((OROWS * 16) // 128, 128)
        h, pin, pout = _mlp_call(
            h, agg4,
            jnp.kron(eye4, gW1[i]), jnp.tile(gb1[i], 4).reshape(1, 128),
            jnp.kron(eye4, gW2[i]), jnp.tile(gb2[i], 4).reshape(1, 128))
        if i == 0:
            p0 = pin
        pooled.append(pout)

    pf = jnp.concatenate([p0] + pooled, axis=1)
    return _score_call(pf, jnp.tile(pW, (1, 4, 1)).reshape(L * 128, OUT),
                       pb)


# speedup vs baseline: 1.3398x; 1.3398x over previous
import functools
import math

import jax
import jax.numpy as jnp
from jax import lax
from jax.experimental import pallas as pl
from jax.experimental.pallas import tpu as pltpu
from jax.experimental.pallas import tpu_sc as plsc

N = 100000
E = 1600000
VOCAB = 101
D = 32
H = 32
OUT = 32
L = 6

_BN = 1.0 / math.sqrt(1.0 + 1e-5)

NC = 2
NS = 16
ROW = 128
CH = 6
G = 132
EPT = CH * ROW * G
E_PAD = EPT * NS
EB = E_PAD // ROW
NZ_PAD = 16 * 6272
ZPT = NZ_PAD // NS
ZCH = 448
OROWS = 2 * N + 256
SROWS = NZ_PAD // ROW
SPT = SROWS // NS


def _sc_agg_body(h2, src2, dst2, scat2, out, src_v, dst_v, rows_v, agg_sh,
                 gsem_a, gsem_b, ssem_a, ssem_b,
                 isrc0, isrc1, idst0, idst1):
    c = lax.axis_index("c")
    s = lax.axis_index("s")

    def _zb(i, _):
        rows_v[0, i, :] = jnp.zeros((16,), jnp.float32)
        return 0
    lax.fori_loop(0, ZCH, _zb, 0)
    z0 = s * ZPT
    for z in range(ZPT // ZCH):
        pltpu.sync_copy(rows_v.at[0, pl.ds(0, ZCH)],
                        agg_sh.at[pl.ds(z0 + z * ZCH, ZCH)])
    plsc.subcore_barrier()

    def fire_src_idx(buf, g, sem):
        base = (s * G + g) * CH
        pltpu.async_copy(src2.at[c, pl.ds(base, CH)], src_v.at[buf], sem)

    def fire_dst_idx(buf, g, sem):
        base = (s * G + g) * CH
        pltpu.async_copy(dst2.at[pl.ds(base, CH)], dst_v.at[buf], sem)

    def wait_src_idx(buf, sem):
        pltpu.make_async_copy(src2.at[c, pl.ds(0, CH)],
                              src_v.at[buf], sem).wait()

    def wait_dst_idx(buf, sem):
        pltpu.make_async_copy(dst2.at[pl.ds(0, CH)],
                              dst_v.at[buf], sem).wait()

    def fire_gathers(buf, sem):
        for j in range(CH):
            pltpu.async_copy(h2.at[src_v.at[buf, j]],
                             rows_v.at[buf, pl.ds(j * ROW, ROW)], sem)

    def drain_gathers(buf, sem):
        pltpu.make_async_copy(h2.at[pl.ds(0, CH * ROW)],
                              rows_v.at[buf], sem).wait()

    def fire_scatters(buf, sem):
        for j in range(CH):
            pltpu.async_copy(rows_v.at[buf, pl.ds(j * ROW, ROW)],
                             agg_sh.at[dst_v.at[buf, j]], sem, add=True)

    def drain_bytes(sem, rows):
        pltpu.make_async_copy(h2.at[pl.ds(0, rows)],
                              agg_sh.at[pl.ds(0, rows)], sem).wait()

    T = G // 2
    fire_src_idx(0, 0, isrc0)
    fire_dst_idx(0, 0, idst0)
    fire_src_idx(1, 1, isrc1)
    fire_dst_idx(1, 1, idst1)
    wait_src_idx(0, isrc0)
    wait_dst_idx(0, idst0)
    wait_src_idx(1, isrc1)
    wait_dst_idx(1, idst1)

    def _eb(t, _):
        @pl.when(t > 0)
        def _():
            drain_bytes(ssem_a, CH * ROW)
            fire_dst_idx(0, 2 * t, idst0)
            wait_src_idx(0, isrc0)
            drain_gathers(1, gsem_b)
            fire_src_idx(1, 2 * t + 1, isrc1)

        @pl.when(t > 1)
        def _():
            wait_dst_idx(1, idst1)

        @pl.when(t > 0)
        def _():
            fire_scatters(1, ssem_b)
        fire_gathers(0, gsem_a)

        @pl.when(t > 0)
        def _():
            drain_bytes(ssem_b, CH * ROW)
            fire_dst_idx(1, 2 * t + 1, idst1)
        drain_gathers(0, gsem_a)

        @pl.when(t < T - 1)
        def _():
            fire_src_idx(0, 2 * t + 2, isrc0)

        @pl.when(t > 0)
        def _():
            wait_src_idx(1, isrc1)
        fire_gathers(1, gsem_b)

        @pl.when(t > 0)
        def _():
            wait_dst_idx(0, idst0)
        fire_scatters(0, ssem_a)
        return 0
    lax.fori_loop(0, T, _eb, 0)

    drain_bytes(ssem_a, CH * ROW)
    drain_gathers(1, gsem_b)
    wait_dst_idx(1, idst1)
    fire_scatters(1, ssem_b)
    drain_bytes(ssem_b, CH * ROW)
    plsc.subcore_barrier()

    rb = s * ZPT
    ib = s * SPT
    for z in range(8):
        b = z % 2
        if z >= 2:
            drain_bytes((ssem_a, ssem_b)[b], CH * ROW)
        pltpu.sync_copy(scat2.at[c, pl.ds(ib + z * CH, CH)], src_v.at[b])
        pltpu.sync_copy(agg_sh.at[pl.ds(rb + z * CH * ROW, CH * ROW)],
                        rows_v.at[b])
        for j in range(CH):
            pltpu.async_copy(rows_v.at[b, pl.ds(j * ROW, ROW)],
                             out.at[src_v.at[b, j]], (ssem_a, ssem_b)[b])
    drain_bytes(ssem_a, CH * ROW)
    pltpu.sync_copy(scat2.at[c, pl.ds(ib + 48, 1)], src_v.at[0, pl.ds(0, 1)])
    pltpu.sync_copy(agg_sh.at[pl.ds(rb + 48 * ROW, ROW)],
                    rows_v.at[0, pl.ds(0, ROW)])
    pltpu.async_copy(rows_v.at[0, pl.ds(0, ROW)],
                     out.at[src_v.at[0, 0]], ssem_a)
    drain_bytes(ssem_b, CH * ROW)
    drain_bytes(ssem_a, ROW)


@functools.cache
def _sc_agg_kernel():
    return pl.kernel(
        _sc_agg_body,
        out_type=jax.ShapeDtypeStruct((OROWS, 16), jnp.float32),
        mesh=plsc.VectorSubcoreMesh(core_axis_name="c", subcore_axis_name="s",
                                    num_cores=NC, num_subcores=NS),
        compiler_params=pltpu.CompilerParams(use_tc_tiling_on_sc=False),
        scratch_types=[
            pltpu.VMEM((2, CH, ROW), jnp.int32),
            pltpu.VMEM((2, CH, ROW), jnp.int32),
            pltpu.VMEM((2, CH * ROW, 16), jnp.float32),
            pltpu.VMEM_SHARED((NZ_PAD, 16), jnp.float32),
            pltpu.SemaphoreType.DMA,
            pltpu.SemaphoreType.DMA,
            pltpu.SemaphoreType.DMA,
            pltpu.SemaphoreType.DMA,
            pltpu.SemaphoreType.DMA,
            pltpu.SemaphoreType.DMA,
            pltpu.SemaphoreType.DMA,
            pltpu.SemaphoreType.DMA,
        ],
    )


def _sc_agg(h2, src2, dst2, scat2):
    return _sc_agg_kernel()(h2, src2, dst2, scat2)


NB = 20000
NB4 = NB // 4


def _embed_body(nt_ref, emb_ref, h_ref):
    ids = nt_ref[...]
    parts = []
    for sl in range(4):
        iota = lax.broadcasted_iota(jnp.int32, (NB4, 128), 1)
        oh = (ids[:, sl:sl + 1] == iota).astype(jnp.float32)
        parts.append(jnp.dot(oh, emb_ref[...],
                             preferred_element_type=jnp.float32))
    h_ref[...] = jnp.concatenate(parts, axis=1)


def _mlp_body(h_ref, agg_ref, w1_ref, b1_ref, w2_ref, b2_ref,
              hn_ref, pin_ref, pout_ref):
    i = pl.program_id(0)
    h = h_ref[...]
    rst = h + agg_ref[...]
    t = jnp.maximum(
        jnp.dot(rst, w1_ref[...], preferred_element_type=jnp.float32)
        + b1_ref[...], 0.0) * _BN
    m = jnp.dot(t, w2_ref[...], preferred_element_type=jnp.float32) \
        + b2_ref[...]
    y = jnp.maximum(m, 0.0) * (_BN * _BN)
    hn_ref[...] = y

    @pl.when(i == 0)
    def _():
        pin_ref[...] = jnp.zeros_like(pin_ref)
        pout_ref[...] = jnp.zeros_like(pout_ref)

    pin_ref[...] += jnp.sum(h, axis=0, keepdims=True)
    pout_ref[...] += jnp.sum(y, axis=0, keepdims=True)


IDXB = 1584


def _idx_body(src_ref, dst_ref, src2_ref, dst2_ref):
    i = pl.program_id(0)
    rows = lax.broadcasted_iota(jnp.int32, (IDXB, 128), 0) + i * IDXB
    cols = lax.broadcasted_iota(jnp.int32, (IDXB, 128), 1)
    gid = rows * 128 + cols
    valid = gid < E
    sv = jnp.where(valid, src_ref[0], gid & 0xFFF)
    src2_ref[0] = 2 * sv
    src2_ref[1] = 2 * sv + 1
    dst2_ref[...] = jnp.where(valid, dst_ref[0], N + (gid & 15))


_idx_call = pl.pallas_call(
    _idx_body,
    grid=(EB // IDXB,),
    in_specs=[
        pl.BlockSpec((1, IDXB, 128), lambda i: (0, i, 0)),
        pl.BlockSpec((1, IDXB, 128), lambda i: (1, i, 0)),
    ],
    out_specs=[
        pl.BlockSpec((NC, IDXB, 128), lambda i: (0, i, 0)),
        pl.BlockSpec((IDXB, 128), lambda i: (i, 0)),
    ],
    out_shape=[
        jax.ShapeDtypeStruct((NC, EB, 128), jnp.int32),
        jax.ShapeDtypeStruct((EB, 128), jnp.int32),
    ],
)


def _score_body(pf_ref, pw_ref, pb_ref, out_ref):
    out_ref[...] = (
        jnp.dot(pf_ref[...], pw_ref[...], preferred_element_type=jnp.float32)
        + jnp.sum(pb_ref[...], axis=0, keepdims=True))


_embed_call = pl.pallas_call(
    _embed_body,
    grid=(N // NB,),
    in_specs=[
        pl.BlockSpec((NB4, 4), lambda i: (i, 0)),
        pl.BlockSpec((128, D), lambda i: (0, 0)),
    ],
    out_specs=pl.BlockSpec((NB4, 128), lambda i: (i, 0)),
    out_shape=jax.ShapeDtypeStruct((N // 4, 128), jnp.float32),
)

_mlp_call = pl.pallas_call(
    _mlp_body,
    grid=(N // NB,),
    in_specs=[
        pl.BlockSpec((NB4, 128), lambda i: (i, 0)),
        pl.BlockSpec((NB4, 128), lambda i: (i, 0)),
        pl.BlockSpec((128, 128), lambda i: (0, 0)),
        pl.BlockSpec((1, 128), lambda i: (0, 0)),
        pl.BlockSpec((128, 128), lambda i: (0, 0)),
        pl.BlockSpec((1, 128), lambda i: (0, 0)),
    ],
    out_specs=[
        pl.BlockSpec((NB4, 128), lambda i: (i, 0)),
        pl.BlockSpec((1, 128), lambda i: (0, 0)),
        pl.BlockSpec((1, 128), lambda i: (0, 0)),
    ],
    out_shape=[
        jax.ShapeDtypeStruct((N // 4, 128), jnp.float32),
        jax.ShapeDtypeStruct((1, 128), jnp.float32),
        jax.ShapeDtypeStruct((1, 128), jnp.float32),
    ],
)

_score_call = pl.pallas_call(
    _score_body,
    in_specs=[
        pl.BlockSpec((1, L * 128), lambda: (0, 0)),
        pl.BlockSpec((L * 128, OUT), lambda: (0, 0)),
        pl.BlockSpec((L, OUT), lambda: (0, 0)),
    ],
    out_specs=pl.BlockSpec((1, OUT), lambda: (0, 0)),
    out_shape=jax.ShapeDtypeStruct((1, OUT), jnp.float32),
)


def kernel(node_type, edge_index, emb, gW1, gb1, gW2, gb2, pW, pb):
    e3 = edge_index.astype(jnp.int32).reshape(2, E // 128, 128)
    src2, dst2 = _idx_call(e3, e3)

    k = jnp.arange(NZ_PAD, dtype=jnp.int32)
    scat2 = jnp.stack(
        [jnp.where(k < N, 2 * k + c, 2 * N + (2 * k + c) % 256)
         for c in range(NC)]).reshape(NC, SROWS, ROW)

    emb_pad = jnp.zeros((128, D), jnp.float32).at[:VOCAB].set(emb)
    h = _embed_call(node_type.astype(jnp.int32).reshape(N // 4, 4), emb_pad)

    eye4 = jnp.eye(4, dtype=jnp.float32)
    pooled = []
    p0 = None
    for i in range(L - 1):
        agg = _sc_agg(h.reshape(2 * N, 16), src2, dst2, scat2)
        agg4 = agg.reshape((OROWS * 16) // 128, 128)
        h, pin, pout = _mlp_call(
            h, agg4,
            jnp.kron(eye4, gW1[i]), jnp.tile(gb1[i], 4).reshape(1, 128),
            jnp.kron(eye4, gW2[i]), jnp.tile(gb2[i], 4).reshape(1, 128))
        if i == 0:
            p0 = pin
        pooled.append(pout)

    pf = jnp.concatenate([p0] + pooled, axis=1)
    return _score_call(pf, jnp.tile(pW, (1, 4, 1)).reshape(L * 128, OUT),
                       pb)
